# baseline (device time: 67625 ns/iter reference)
import jax
import jax.numpy as jnp
from jax import lax
from jax.experimental import pallas as pl
from jax.experimental.pallas import tpu as pltpu

N_DEV = 32
N_R = 16
N_L = 15


def kernel(x, w_mat):
    m_per, k = x.shape
    _, n_per = w_mat.shape
    M = N_DEV * m_per

    def body(x_ref, w_ref, out_ref, xg_ref, send_r, recv_r, send_l, recv_l):
        me = lax.axis_index("i")
        right = lax.rem(me + 1, N_DEV)
        left = lax.rem(me + N_DEV - 1, N_DEV)

        barrier_sem = pltpu.get_barrier_semaphore()
        for nbr in (left, right):
            pl.semaphore_signal(
                barrier_sem, inc=1,
                device_id=(nbr,), device_id_type=pl.DeviceIdType.MESH,
            )
        pl.semaphore_wait(barrier_sem, 2)

        def rows(origin):
            o = lax.rem(origin + 2 * N_DEV, N_DEV)
            return (pl.ds(o * m_per, m_per), slice(None))

        xg_ref[rows(me)] = x_ref[...]

        sends = []

        def start_send(h, direction):
            if direction > 0:
                origin, tgt, ssem, rsem = me - h, right, send_r, recv_r
            else:
                origin, tgt, ssem, rsem = me + h, left, send_l, recv_l
            src = x_ref.at[:, :] if h == 0 else xg_ref.at[rows(origin)]
            rd = pltpu.make_async_remote_copy(
                src_ref=src,
                dst_ref=xg_ref.at[rows(origin)],
                send_sem=ssem.at[h],
                recv_sem=rsem.at[h],
                device_id=(tgt,),
                device_id_type=pl.DeviceIdType.MESH,
            )
            rd.start()
            sends.append(rd)

        def wait_recv(h, direction):
            if direction > 0:
                origin, rsem = me - 1 - h, recv_r
            else:
                origin, rsem = me + 1 + h, recv_l
            rd = pltpu.make_async_remote_copy(
                src_ref=x_ref.at[:, :],
                dst_ref=xg_ref.at[rows(origin)],
                send_sem=rsem.at[h],
                recv_sem=rsem.at[h],
                device_id=(right,),
                device_id_type=pl.DeviceIdType.MESH,
            )
            rd.wait_recv()

        start_send(0, +1)
        start_send(0, -1)
        for h in range(1, N_R):
            wait_recv(h - 1, +1)
            start_send(h, +1)
            if h < N_L:
                wait_recv(h - 1, -1)
                start_send(h, -1)
        wait_recv(N_R - 1, +1)
        wait_recv(N_L - 1, -1)
        for rd in sends:
            rd.wait_send()

        y = jnp.dot(xg_ref[...], w_ref[...], preferred_element_type=jnp.float32)
        c = 0.7978845608028654
        out_ref[...] = 0.5 * y * (1.0 + jnp.tanh(c * (y + 0.044715 * y * y * y)))

    return pl.pallas_call(
        body,
        out_shape=jax.ShapeDtypeStruct((M, n_per), jnp.float32),
        in_specs=[
            pl.BlockSpec(memory_space=pltpu.VMEM),
            pl.BlockSpec(memory_space=pltpu.VMEM),
        ],
        out_specs=pl.BlockSpec(memory_space=pltpu.VMEM),
        scratch_shapes=[
            pltpu.VMEM((M, k), jnp.float32),
            pltpu.SemaphoreType.DMA((N_R,)),
            pltpu.SemaphoreType.DMA((N_R,)),
            pltpu.SemaphoreType.DMA((N_L,)),
            pltpu.SemaphoreType.DMA((N_L,)),
        ],
        compiler_params=pltpu.CompilerParams(collective_id=0),
    )(x, w_mat)


# device time: 42153 ns/iter; 1.6043x vs baseline; 1.6043x over previous
import jax
import jax.numpy as jnp
from jax import lax
from jax.experimental import pallas as pl
from jax.experimental.pallas import tpu as pltpu

N_DEV = 32
N_PLANE = 8
N_Z = 4


def kernel(x, w_mat):
    m_per, k = x.shape
    _, n_per = w_mat.shape
    M = N_DEV * m_per

    def body(x_ref, w_ref, out_ref, xg_ref,
             su, ru, sd, rdn, sR, rR, sL, rL):
        me = lax.axis_index("i")
        p = me // N_PLANE
        q = lax.rem(me, N_PLANE)
        base = me - q

        def ring_q(rr):
            return jnp.where(rr < 3, rr, jnp.where(rr < 6, rr + 2, 10 - rr))

        r = jnp.where(q < 3, q, jnp.where(q >= 5, q - 2,
                                          jnp.where(q == 4, 6, 7)))
        right_dev = base + ring_q(lax.rem(r + 1, 8))
        left_dev = base + ring_q(lax.rem(r + 7, 8))
        up = me + N_PLANE
        down = me - N_PLANE
        has_up = p < N_Z - 1
        has_down = p > 0

        INC = 1 << 15
        barrier_sem = pltpu.get_barrier_semaphore()
        for nbr in (left_dev, right_dev):
            pl.semaphore_signal(
                barrier_sem, inc=INC,
                device_id=(nbr,), device_id_type=pl.DeviceIdType.MESH,
            )

        @pl.when(has_up)
        def _():
            pl.semaphore_signal(
                barrier_sem, inc=INC,
                device_id=(up,), device_id_type=pl.DeviceIdType.MESH,
            )

        @pl.when(has_down)
        def _():
            pl.semaphore_signal(
                barrier_sem, inc=INC,
                device_id=(down,), device_id_type=pl.DeviceIdType.MESH,
            )

        pl.semaphore_wait(barrier_sem, 2 * INC)

        @pl.when(has_up)
        def _():
            pl.semaphore_wait(barrier_sem, INC)

        @pl.when(has_down)
        def _():
            pl.semaphore_wait(barrier_sem, INC)

        def slot_at(s):
            return (pl.ds(s * m_per, m_per), slice(None))

        def rdma(src, dst_slot, ssem, rsem, dev):
            return pltpu.make_async_remote_copy(
                src_ref=src,
                dst_ref=xg_ref.at[slot_at(dst_slot)],
                send_sem=ssem,
                recv_sem=rsem,
                device_id=(dev,),
                device_id_type=pl.DeviceIdType.MESH,
            )

        sends = []

        def cstart(cond, rd):
            if cond is None:
                rd.start()
            else:
                @pl.when(cond)
                def _():
                    rd.start()
            sends.append((cond, rd))

        def cwait_recv(cond, rd):
            if cond is None:
                rd.wait_recv()
            else:
                @pl.when(cond)
                def _():
                    rd.wait_recv()

        own_slot = q * N_Z + p
        xg_ref[slot_at(own_slot)] = x_ref[...]

        x_src = x_ref.at[:, :]

        cstart(has_up, rdma(x_src, own_slot, su.at[0], ru.at[0], up))
        cstart(has_down, rdma(x_src, own_slot, sd.at[0], rdn.at[0], down))

        for d in (1, 2, 3):
            cond_u = p >= d
            sj = q * N_Z + (p - d)
            cwait_recv(cond_u, rdma(x_src, sj, ru.at[d - 1], ru.at[d - 1], up))
            if d <= 2:
                cstart(jnp.logical_and(cond_u, has_up),
                       rdma(xg_ref.at[slot_at(sj)], sj, su.at[d], ru.at[d], up))

            cond_d = p + d <= N_Z - 1
            sj = q * N_Z + (p + d)
            cwait_recv(cond_d, rdma(x_src, sj, rdn.at[d - 1], rdn.at[d - 1], up))
            if d <= 2:
                cstart(jnp.logical_and(cond_d, has_down),
                       rdma(xg_ref.at[slot_at(sj)], sj, sd.at[d], rdn.at[d], down))

        for j in range(N_Z):
            sj = q * N_Z + j
            cstart(None, rdma(xg_ref.at[slot_at(sj)], sj,
                              sR.at[j], rR.at[j], right_dev))
            cstart(None, rdma(xg_ref.at[slot_at(sj)], sj,
                              sL.at[j], rL.at[j], left_dev))

        for h in (1, 2, 3):
            q_or = ring_q(lax.rem(r - h + 8, 8))
            for j in range(N_Z):
                sj = q_or * N_Z + j
                cwait_recv(None, rdma(x_src, sj, rR.at[(h - 1) * N_Z + j],
                                      rR.at[(h - 1) * N_Z + j], right_dev))
                cstart(None, rdma(xg_ref.at[slot_at(sj)], sj,
                                  sR.at[h * N_Z + j], rR.at[h * N_Z + j],
                                  right_dev))
            if h <= 2:
                q_ol = ring_q(lax.rem(r + h, 8))
                for j in range(N_Z):
                    sj = q_ol * N_Z + j
                    cwait_recv(None, rdma(x_src, sj, rL.at[(h - 1) * N_Z + j],
                                          rL.at[(h - 1) * N_Z + j], left_dev))
                    cstart(None, rdma(xg_ref.at[slot_at(sj)], sj,
                                      sL.at[h * N_Z + j], rL.at[h * N_Z + j],
                                      left_dev))

        q_o4 = ring_q(lax.rem(r + 4, 8))
        q_o3l = ring_q(lax.rem(r + 3, 8))
        for j in range(N_Z):
            cwait_recv(None, rdma(x_src, q_o4 * N_Z + j,
                                  rR.at[3 * N_Z + j], rR.at[3 * N_Z + j],
                                  right_dev))
        for j in range(N_Z):
            cwait_recv(None, rdma(x_src, q_o3l * N_Z + j,
                                  rL.at[2 * N_Z + j], rL.at[2 * N_Z + j],
                                  left_dev))

        for cond, rd in sends:
            if cond is None:
                rd.wait_send()
            else:
                @pl.when(cond)
                def _(rd=rd):
                    rd.wait_send()

        c = 0.7978845608028654
        w = w_ref[...]
        for ell in range(N_DEV):
            s = (ell % N_PLANE) * N_Z + ell // N_PLANE
            y = jnp.dot(xg_ref[s * m_per:(s + 1) * m_per, :], w,
                        preferred_element_type=jnp.float32)
            out_ref[ell * m_per:(ell + 1) * m_per, :] = \
                0.5 * y * (1.0 + jnp.tanh(c * (y + 0.044715 * y * y * y)))

    return pl.pallas_call(
        body,
        out_shape=jax.ShapeDtypeStruct((M, n_per), jnp.float32),
        in_specs=[
            pl.BlockSpec(memory_space=pltpu.VMEM),
            pl.BlockSpec(memory_space=pltpu.VMEM),
        ],
        out_specs=pl.BlockSpec(memory_space=pltpu.VMEM),
        scratch_shapes=[
            pltpu.VMEM((M, k), jnp.float32),
            pltpu.SemaphoreType.DMA((3,)),
            pltpu.SemaphoreType.DMA((3,)),
            pltpu.SemaphoreType.DMA((3,)),
            pltpu.SemaphoreType.DMA((3,)),
            pltpu.SemaphoreType.DMA((16,)),
            pltpu.SemaphoreType.DMA((16,)),
            pltpu.SemaphoreType.DMA((12,)),
            pltpu.SemaphoreType.DMA((12,)),
        ],
        compiler_params=pltpu.CompilerParams(collective_id=0),
    )(x, w_mat)


# device time: 37974 ns/iter; 1.7808x vs baseline; 1.1100x over previous
import jax
import jax.numpy as jnp
from jax import lax
from jax.experimental import pallas as pl
from jax.experimental.pallas import tpu as pltpu

N_DEV = 32
N_PLANE = 8
N_Z = 4


def kernel(x, w_mat):
    m_per, k = x.shape
    _, n_per = w_mat.shape
    M = N_DEV * m_per

    def body(x_ref, w_ref, out_ref, xg_ref,
             su, ru, sd, rdn, sR, rR, sL, rL):
        me = lax.axis_index("i")
        p = me // N_PLANE
        q = lax.rem(me, N_PLANE)
        base = me - q

        def ring_q(rr):
            return jnp.where(rr < 3, rr, jnp.where(rr < 6, rr + 2, 10 - rr))

        r = jnp.where(q < 3, q, jnp.where(q >= 5, q - 2,
                                          jnp.where(q == 4, 6, 7)))
        right_dev = base + ring_q(lax.rem(r + 1, 8))
        left_dev = base + ring_q(lax.rem(r + 7, 8))
        up = me + N_PLANE
        down = me - N_PLANE
        has_up = p < N_Z - 1
        has_down = p > 0

        INC = 1 << 15
        barrier_sem = pltpu.get_barrier_semaphore()
        for nbr in (left_dev, right_dev):
            pl.semaphore_signal(
                barrier_sem, inc=INC,
                device_id=(nbr,), device_id_type=pl.DeviceIdType.MESH,
            )

        @pl.when(has_up)
        def _():
            pl.semaphore_signal(
                barrier_sem, inc=INC,
                device_id=(up,), device_id_type=pl.DeviceIdType.MESH,
            )

        @pl.when(has_down)
        def _():
            pl.semaphore_signal(
                barrier_sem, inc=INC,
                device_id=(down,), device_id_type=pl.DeviceIdType.MESH,
            )

        pl.semaphore_wait(barrier_sem, 2 * INC)

        @pl.when(has_up)
        def _():
            pl.semaphore_wait(barrier_sem, INC)

        @pl.when(has_down)
        def _():
            pl.semaphore_wait(barrier_sem, INC)

        def slot_at(s):
            return (pl.ds(s * m_per, m_per), slice(None))

        def rdma(src, dst_slot, ssem, rsem, dev):
            return pltpu.make_async_remote_copy(
                src_ref=src,
                dst_ref=xg_ref.at[slot_at(dst_slot)],
                send_sem=ssem,
                recv_sem=rsem,
                device_id=(dev,),
                device_id_type=pl.DeviceIdType.MESH,
            )

        sends = []

        def cstart(cond, rd):
            if cond is None:
                rd.start()
            else:
                @pl.when(cond)
                def _():
                    rd.start()
            sends.append((cond, rd))

        def cwait_recv(cond, rd):
            if cond is None:
                rd.wait_recv()
            else:
                @pl.when(cond)
                def _():
                    rd.wait_recv()

        own_slot = q * N_Z + p
        xg_ref[slot_at(own_slot)] = x_ref[...]

        x_src = x_ref.at[:, :]

        cstart(has_up, rdma(x_src, own_slot, su.at[0], ru.at[0], up))
        cstart(has_down, rdma(x_src, own_slot, sd.at[0], rdn.at[0], down))
        cstart(None, rdma(x_src, own_slot, sR.at[p], rR.at[p], right_dev))
        cstart(None, rdma(x_src, own_slot, sL.at[p], rL.at[p], left_dev))

        for d in (1, 2, 3):
            ju = p - d
            cond_u = p >= d
            sj = q * N_Z + ju
            cwait_recv(cond_u, rdma(x_src, sj, ru.at[d - 1], ru.at[d - 1], up))
            if d <= 2:
                cstart(jnp.logical_and(cond_u, has_up),
                       rdma(xg_ref.at[slot_at(sj)], sj, su.at[d], ru.at[d], up))
            cstart(cond_u, rdma(xg_ref.at[slot_at(sj)], sj,
                                sR.at[ju], rR.at[ju], right_dev))
            cstart(cond_u, rdma(xg_ref.at[slot_at(sj)], sj,
                                sL.at[ju], rL.at[ju], left_dev))

            jd = p + d
            cond_d = jd <= N_Z - 1
            sj = q * N_Z + jd
            cwait_recv(cond_d, rdma(x_src, sj, rdn.at[d - 1], rdn.at[d - 1], up))
            if d <= 2:
                cstart(jnp.logical_and(cond_d, has_down),
                       rdma(xg_ref.at[slot_at(sj)], sj, sd.at[d], rdn.at[d], down))
            cstart(cond_d, rdma(xg_ref.at[slot_at(sj)], sj,
                                sR.at[jd], rR.at[jd], right_dev))
            cstart(cond_d, rdma(xg_ref.at[slot_at(sj)], sj,
                                sL.at[jd], rL.at[jd], left_dev))

        for h in (1, 2, 3):
            q_or = ring_q(lax.rem(r - h + 8, 8))
            for j in range(N_Z):
                sj = q_or * N_Z + j
                cwait_recv(None, rdma(x_src, sj, rR.at[(h - 1) * N_Z + j],
                                      rR.at[(h - 1) * N_Z + j], right_dev))
                cstart(None, rdma(xg_ref.at[slot_at(sj)], sj,
                                  sR.at[h * N_Z + j], rR.at[h * N_Z + j],
                                  right_dev))
            if h <= 2:
                q_ol = ring_q(lax.rem(r + h, 8))
                for j in range(N_Z):
                    sj = q_ol * N_Z + j
                    cwait_recv(None, rdma(x_src, sj, rL.at[(h - 1) * N_Z + j],
                                          rL.at[(h - 1) * N_Z + j], left_dev))
                    cstart(None, rdma(xg_ref.at[slot_at(sj)], sj,
                                      sL.at[h * N_Z + j], rL.at[h * N_Z + j],
                                      left_dev))

        q_o4 = ring_q(lax.rem(r + 4, 8))
        q_o3l = ring_q(lax.rem(r + 3, 8))
        for j in range(N_Z):
            cwait_recv(None, rdma(x_src, q_o4 * N_Z + j,
                                  rR.at[3 * N_Z + j], rR.at[3 * N_Z + j],
                                  right_dev))
        for j in range(N_Z):
            cwait_recv(None, rdma(x_src, q_o3l * N_Z + j,
                                  rL.at[2 * N_Z + j], rL.at[2 * N_Z + j],
                                  left_dev))

        for cond, rd in sends:
            if cond is None:
                rd.wait_send()
            else:
                @pl.when(cond)
                def _(rd=rd):
                    rd.wait_send()

        c = 0.7978845608028654
        w = w_ref[...]
        for ell in range(N_DEV):
            s = (ell % N_PLANE) * N_Z + ell // N_PLANE
            y = jnp.dot(xg_ref[s * m_per:(s + 1) * m_per, :], w,
                        preferred_element_type=jnp.float32)
            out_ref[ell * m_per:(ell + 1) * m_per, :] = \
                0.5 * y * (1.0 + jnp.tanh(c * (y + 0.044715 * y * y * y)))

    return pl.pallas_call(
        body,
        out_shape=jax.ShapeDtypeStruct((M, n_per), jnp.float32),
        in_specs=[
            pl.BlockSpec(memory_space=pltpu.VMEM),
            pl.BlockSpec(memory_space=pltpu.VMEM),
        ],
        out_specs=pl.BlockSpec(memory_space=pltpu.VMEM),
        scratch_shapes=[
            pltpu.VMEM((M, k), jnp.float32),
            pltpu.SemaphoreType.DMA((3,)),
            pltpu.SemaphoreType.DMA((3,)),
            pltpu.SemaphoreType.DMA((3,)),
            pltpu.SemaphoreType.DMA((3,)),
            pltpu.SemaphoreType.DMA((16,)),
            pltpu.SemaphoreType.DMA((16,)),
            pltpu.SemaphoreType.DMA((12,)),
            pltpu.SemaphoreType.DMA((12,)),
        ],
        compiler_params=pltpu.CompilerParams(collective_id=0),
    )(x, w_mat)


# device time: 37950 ns/iter; 1.7819x vs baseline; 1.0006x over previous
import jax
import jax.numpy as jnp
from jax import lax
from jax.experimental import pallas as pl
from jax.experimental.pallas import tpu as pltpu

N_DEV = 32
N_PLANE = 8
N_Z = 4


def kernel(x, w_mat):
    m_per, k = x.shape
    _, n_per = w_mat.shape
    M = N_DEV * m_per

    def body(x_ref, w_ref, out_ref, xg_ref,
             su, ru, sd, rdn, sR, rR, sL, rL):
        me = lax.axis_index("i")
        p = me // N_PLANE
        q = lax.rem(me, N_PLANE)
        base = me - q

        def ring_q(rr):
            return jnp.where(rr < 3, rr, jnp.where(rr < 6, rr + 2, 10 - rr))

        r = jnp.where(q < 3, q, jnp.where(q >= 5, q - 2,
                                          jnp.where(q == 4, 6, 7)))
        right_dev = base + ring_q(lax.rem(r + 1, 8))
        left_dev = base + ring_q(lax.rem(r + 7, 8))
        up = me + N_PLANE
        down = me - N_PLANE
        has_up = p < N_Z - 1
        has_down = p > 0

        INC = 1 << 15
        barrier_sem = pltpu.get_barrier_semaphore()
        for nbr in (left_dev, right_dev):
            pl.semaphore_signal(
                barrier_sem, inc=INC,
                device_id=(nbr,), device_id_type=pl.DeviceIdType.MESH,
            )

        @pl.when(has_up)
        def _():
            pl.semaphore_signal(
                barrier_sem, inc=INC,
                device_id=(up,), device_id_type=pl.DeviceIdType.MESH,
            )

        @pl.when(has_down)
        def _():
            pl.semaphore_signal(
                barrier_sem, inc=INC,
                device_id=(down,), device_id_type=pl.DeviceIdType.MESH,
            )

        pl.semaphore_wait(barrier_sem, 2 * INC)

        @pl.when(has_up)
        def _():
            pl.semaphore_wait(barrier_sem, INC)

        @pl.when(has_down)
        def _():
            pl.semaphore_wait(barrier_sem, INC)

        def slot_at(s):
            return (pl.ds(s * m_per, m_per), slice(None))

        def rdma(src, dst_slot, ssem, rsem, dev):
            return pltpu.make_async_remote_copy(
                src_ref=src,
                dst_ref=xg_ref.at[slot_at(dst_slot)],
                send_sem=ssem,
                recv_sem=rsem,
                device_id=(dev,),
                device_id_type=pl.DeviceIdType.MESH,
            )

        sends = []

        def cstart(cond, rd):
            if cond is None:
                rd.start()
            else:
                @pl.when(cond)
                def _():
                    rd.start()
            sends.append((cond, rd))

        def cwait_recv(cond, rd):
            if cond is None:
                rd.wait_recv()
            else:
                @pl.when(cond)
                def _():
                    rd.wait_recv()

        own_slot = q * N_Z + p
        xg_ref[slot_at(own_slot)] = x_ref[...]

        x_src = x_ref.at[:, :]

        cstart(has_up, rdma(x_src, own_slot, su.at[0], ru.at[0], up))
        cstart(has_down, rdma(x_src, own_slot, sd.at[0], rdn.at[0], down))
        cstart(None, rdma(x_src, own_slot, sR.at[p], rR.at[p], right_dev))
        cstart(None, rdma(x_src, own_slot, sL.at[p], rL.at[p], left_dev))

        for d in (1, 2, 3):
            ju = p - d
            cond_u = p >= d
            sj = q * N_Z + ju
            cwait_recv(cond_u, rdma(x_src, sj, ru.at[d - 1], ru.at[d - 1], up))
            if d <= 2:
                cstart(jnp.logical_and(cond_u, has_up),
                       rdma(xg_ref.at[slot_at(sj)], sj, su.at[d], ru.at[d], up))
            cstart(cond_u, rdma(xg_ref.at[slot_at(sj)], sj,
                                sR.at[ju], rR.at[ju], right_dev))
            cstart(cond_u, rdma(xg_ref.at[slot_at(sj)], sj,
                                sL.at[ju], rL.at[ju], left_dev))

            jd = p + d
            cond_d = jd <= N_Z - 1
            sj = q * N_Z + jd
            cwait_recv(cond_d, rdma(x_src, sj, rdn.at[d - 1], rdn.at[d - 1], up))
            if d <= 2:
                cstart(jnp.logical_and(cond_d, has_down),
                       rdma(xg_ref.at[slot_at(sj)], sj, sd.at[d], rdn.at[d], down))
            cstart(cond_d, rdma(xg_ref.at[slot_at(sj)], sj,
                                sR.at[jd], rR.at[jd], right_dev))
            cstart(cond_d, rdma(xg_ref.at[slot_at(sj)], sj,
                                sL.at[jd], rL.at[jd], left_dev))

        def for_each_chunk(f):
            f(None, p)
            for d in (1, 2, 3):
                f(p >= d, p - d)
                f(p + d <= N_Z - 1, p + d)

        for h in (1, 2, 3):
            q_or = ring_q(lax.rem(r - h + 8, 8))
            q_ol = ring_q(lax.rem(r + h, 8)) if h <= 2 else None

            def fwd(cond, j, h=h, q_or=q_or, q_ol=q_ol):
                sj = q_or * N_Z + j
                cwait_recv(cond, rdma(x_src, sj, rR.at[(h - 1) * N_Z + j],
                                      rR.at[(h - 1) * N_Z + j], right_dev))
                cstart(cond, rdma(xg_ref.at[slot_at(sj)], sj,
                                  sR.at[h * N_Z + j], rR.at[h * N_Z + j],
                                  right_dev))
                if q_ol is not None:
                    sl = q_ol * N_Z + j
                    cwait_recv(cond, rdma(x_src, sl, rL.at[(h - 1) * N_Z + j],
                                          rL.at[(h - 1) * N_Z + j], left_dev))
                    cstart(cond, rdma(xg_ref.at[slot_at(sl)], sl,
                                      sL.at[h * N_Z + j], rL.at[h * N_Z + j],
                                      left_dev))

            for_each_chunk(fwd)

        q_o4 = ring_q(lax.rem(r + 4, 8))
        q_o3l = ring_q(lax.rem(r + 3, 8))

        def last(cond, j):
            cwait_recv(cond, rdma(x_src, q_o4 * N_Z + j,
                                  rR.at[3 * N_Z + j], rR.at[3 * N_Z + j],
                                  right_dev))
            cwait_recv(cond, rdma(x_src, q_o3l * N_Z + j,
                                  rL.at[2 * N_Z + j], rL.at[2 * N_Z + j],
                                  left_dev))

        for_each_chunk(last)

        for cond, rd in sends:
            if cond is None:
                rd.wait_send()
            else:
                @pl.when(cond)
                def _(rd=rd):
                    rd.wait_send()

        c = 0.7978845608028654
        w = w_ref[...]
        for ell in range(N_DEV):
            s = (ell % N_PLANE) * N_Z + ell // N_PLANE
            y = jnp.dot(xg_ref[s * m_per:(s + 1) * m_per, :], w,
                        preferred_element_type=jnp.float32)
            out_ref[ell * m_per:(ell + 1) * m_per, :] = \
                0.5 * y * (1.0 + jnp.tanh(c * (y + 0.044715 * y * y * y)))

    return pl.pallas_call(
        body,
        out_shape=jax.ShapeDtypeStruct((M, n_per), jnp.float32),
        in_specs=[
            pl.BlockSpec(memory_space=pltpu.VMEM),
            pl.BlockSpec(memory_space=pltpu.VMEM),
        ],
        out_specs=pl.BlockSpec(memory_space=pltpu.VMEM),
        scratch_shapes=[
            pltpu.VMEM((M, k), jnp.float32),
            pltpu.SemaphoreType.DMA((3,)),
            pltpu.SemaphoreType.DMA((3,)),
            pltpu.SemaphoreType.DMA((3,)),
            pltpu.SemaphoreType.DMA((3,)),
            pltpu.SemaphoreType.DMA((16,)),
            pltpu.SemaphoreType.DMA((16,)),
            pltpu.SemaphoreType.DMA((12,)),
            pltpu.SemaphoreType.DMA((12,)),
        ],
        compiler_params=pltpu.CompilerParams(collective_id=0),
    )(x, w_mat)


# device time: 37416 ns/iter; 1.8074x vs baseline; 1.0143x over previous
import jax
import jax.numpy as jnp
from jax import lax
from jax.experimental import pallas as pl
from jax.experimental.pallas import tpu as pltpu

N_DEV = 32
N_PLANE = 8
N_Z = 4


def kernel(x, w_mat):
    m_per, k = x.shape
    _, n_per = w_mat.shape
    M = N_DEV * m_per

    def body(x_ref, w_ref, out_ref, xg_ref,
             su, ru, sd, rdn, sR, rR, sL, rL):
        me = lax.axis_index("i")
        p = me // N_PLANE
        q = lax.rem(me, N_PLANE)
        base = me - q

        def ring_q(rr):
            return jnp.where(rr < 3, rr, jnp.where(rr < 6, rr + 2, 10 - rr))

        r = jnp.where(q < 3, q, jnp.where(q >= 5, q - 2,
                                          jnp.where(q == 4, 6, 7)))
        right_dev = base + ring_q(lax.rem(r + 1, 8))
        left_dev = base + ring_q(lax.rem(r + 7, 8))
        up = me + N_PLANE
        down = me - N_PLANE
        has_up = p < N_Z - 1
        has_down = p > 0

        INC = 1 << 15
        barrier_sem = pltpu.get_barrier_semaphore()
        for nbr in (left_dev, right_dev):
            pl.semaphore_signal(
                barrier_sem, inc=INC,
                device_id=(nbr,), device_id_type=pl.DeviceIdType.MESH,
            )

        @pl.when(has_up)
        def _():
            pl.semaphore_signal(
                barrier_sem, inc=INC,
                device_id=(up,), device_id_type=pl.DeviceIdType.MESH,
            )

        @pl.when(has_down)
        def _():
            pl.semaphore_signal(
                barrier_sem, inc=INC,
                device_id=(down,), device_id_type=pl.DeviceIdType.MESH,
            )

        pl.semaphore_wait(barrier_sem, 2 * INC)

        @pl.when(has_up)
        def _():
            pl.semaphore_wait(barrier_sem, INC)

        @pl.when(has_down)
        def _():
            pl.semaphore_wait(barrier_sem, INC)

        def slot_at(s):
            return (pl.ds(s * m_per, m_per), slice(None))

        def rdma(src, dst_slot, ssem, rsem, dev):
            return pltpu.make_async_remote_copy(
                src_ref=src,
                dst_ref=xg_ref.at[slot_at(dst_slot)],
                send_sem=ssem,
                recv_sem=rsem,
                device_id=(dev,),
                device_id_type=pl.DeviceIdType.MESH,
            )

        sends = []

        def cstart(cond, rd):
            if cond is None:
                rd.start()
            else:
                @pl.when(cond)
                def _():
                    rd.start()
            sends.append((cond, rd))

        def cwait_recv(cond, rd):
            if cond is None:
                rd.wait_recv()
            else:
                @pl.when(cond)
                def _():
                    rd.wait_recv()

        own_slot = q * N_Z + p
        xg_ref[slot_at(own_slot)] = x_ref[...]

        x_src = x_ref.at[:, :]

        cstart(has_up, rdma(x_src, own_slot, su.at[0], ru.at[0], up))
        cstart(has_down, rdma(x_src, own_slot, sd.at[0], rdn.at[0], down))
        cstart(None, rdma(x_src, own_slot, sR.at[p], rR.at[p], right_dev))
        cstart(None, rdma(x_src, own_slot, sL.at[p], rL.at[p], left_dev))

        for d in (1, 2, 3):
            ju = p - d
            cond_u = p >= d
            sj = q * N_Z + ju
            cwait_recv(cond_u, rdma(x_src, sj, ru.at[d - 1], ru.at[d - 1], up))
            if d <= 2:
                cstart(jnp.logical_and(cond_u, has_up),
                       rdma(xg_ref.at[slot_at(sj)], sj, su.at[d], ru.at[d], up))
            cstart(cond_u, rdma(xg_ref.at[slot_at(sj)], sj,
                                sR.at[ju], rR.at[ju], right_dev))
            cstart(cond_u, rdma(xg_ref.at[slot_at(sj)], sj,
                                sL.at[ju], rL.at[ju], left_dev))

            jd = p + d
            cond_d = jd <= N_Z - 1
            sj = q * N_Z + jd
            cwait_recv(cond_d, rdma(x_src, sj, rdn.at[d - 1], rdn.at[d - 1], up))
            if d <= 2:
                cstart(jnp.logical_and(cond_d, has_down),
                       rdma(xg_ref.at[slot_at(sj)], sj, sd.at[d], rdn.at[d], down))
            cstart(cond_d, rdma(xg_ref.at[slot_at(sj)], sj,
                                sR.at[jd], rR.at[jd], right_dev))
            cstart(cond_d, rdma(xg_ref.at[slot_at(sj)], sj,
                                sL.at[jd], rL.at[jd], left_dev))

        def for_each_chunk(f):
            f(None, p)
            for d in (1, 2, 3):
                f(p >= d, p - d)
                f(p + d <= N_Z - 1, p + d)

        for h in (1, 2, 3):
            q_or = ring_q(lax.rem(r - h + 8, 8))
            q_ol = ring_q(lax.rem(r + h, 8)) if h <= 2 else None

            def fwd(cond, j, h=h, q_or=q_or, q_ol=q_ol):
                sj = q_or * N_Z + j
                cwait_recv(cond, rdma(x_src, sj, rR.at[(h - 1) * N_Z + j],
                                      rR.at[(h - 1) * N_Z + j], right_dev))
                cstart(cond, rdma(xg_ref.at[slot_at(sj)], sj,
                                  sR.at[h * N_Z + j], rR.at[h * N_Z + j],
                                  right_dev))
                if q_ol is not None:
                    sl = q_ol * N_Z + j
                    cwait_recv(cond, rdma(x_src, sl, rL.at[(h - 1) * N_Z + j],
                                          rL.at[(h - 1) * N_Z + j], left_dev))
                    cstart(cond, rdma(xg_ref.at[slot_at(sl)], sl,
                                      sL.at[h * N_Z + j], rL.at[h * N_Z + j],
                                      left_dev))

            for_each_chunk(fwd)

        q_o4 = ring_q(lax.rem(r + 4, 8))
        q_o3l = ring_q(lax.rem(r + 3, 8))

        def last(cond, j):
            cwait_recv(cond, rdma(x_src, q_o4 * N_Z + j,
                                  rR.at[3 * N_Z + j], rR.at[3 * N_Z + j],
                                  right_dev))
            cwait_recv(cond, rdma(x_src, q_o3l * N_Z + j,
                                  rL.at[2 * N_Z + j], rL.at[2 * N_Z + j],
                                  left_dev))

        for_each_chunk(last)

        for cond, rd in sends:
            if cond is None:
                rd.wait_send()
            else:
                @pl.when(cond)
                def _(rd=rd):
                    rd.wait_send()

        y = jnp.dot(xg_ref[...], w_ref[...], preferred_element_type=jnp.float32)
        c = 0.7978845608028654
        g = 0.5 * y * (1.0 + jnp.tanh(c * (y + 0.044715 * y * y * y)))
        for ell in range(N_DEV):
            s = (ell % N_PLANE) * N_Z + ell // N_PLANE
            out_ref[ell * m_per:(ell + 1) * m_per, :] = \
                g[s * m_per:(s + 1) * m_per, :]

    return pl.pallas_call(
        body,
        out_shape=jax.ShapeDtypeStruct((M, n_per), jnp.float32),
        in_specs=[
            pl.BlockSpec(memory_space=pltpu.VMEM),
            pl.BlockSpec(memory_space=pltpu.VMEM),
        ],
        out_specs=pl.BlockSpec(memory_space=pltpu.VMEM),
        scratch_shapes=[
            pltpu.VMEM((M, k), jnp.float32),
            pltpu.SemaphoreType.DMA((3,)),
            pltpu.SemaphoreType.DMA((3,)),
            pltpu.SemaphoreType.DMA((3,)),
            pltpu.SemaphoreType.DMA((3,)),
            pltpu.SemaphoreType.DMA((16,)),
            pltpu.SemaphoreType.DMA((16,)),
            pltpu.SemaphoreType.DMA((12,)),
            pltpu.SemaphoreType.DMA((12,)),
        ],
        compiler_params=pltpu.CompilerParams(collective_id=0),
    )(x, w_mat)


# device time: 36045 ns/iter; 1.8761x vs baseline; 1.0380x over previous
import jax
import jax.numpy as jnp
from jax import lax
from jax.experimental import pallas as pl
from jax.experimental.pallas import tpu as pltpu

N_DEV = 32
N_PLANE = 8
N_Z = 4
N_H = 2


def kernel(x, w_mat):
    m_per, k = x.shape
    _, n_per = w_mat.shape
    M = N_DEV * m_per
    m_half = m_per // N_H

    def body(x_ref, w_ref, out_ref, xg_ref,
             su, ru, sd, rdn, sR, rR, sL, rL):
        me = lax.axis_index("i")
        p = me // N_PLANE
        q = lax.rem(me, N_PLANE)
        base = me - q

        def ring_q(rr):
            return jnp.where(rr < 3, rr, jnp.where(rr < 6, rr + 2, 10 - rr))

        r = jnp.where(q < 3, q, jnp.where(q >= 5, q - 2,
                                          jnp.where(q == 4, 6, 7)))
        right_dev = base + ring_q(lax.rem(r + 1, 8))
        left_dev = base + ring_q(lax.rem(r + 7, 8))
        up = me + N_PLANE
        down = me - N_PLANE
        has_up = p < N_Z - 1
        has_down = p > 0

        INC = 1 << 15
        barrier_sem = pltpu.get_barrier_semaphore()
        for nbr in (left_dev, right_dev):
            pl.semaphore_signal(
                barrier_sem, inc=INC,
                device_id=(nbr,), device_id_type=pl.DeviceIdType.MESH,
            )

        @pl.when(has_up)
        def _():
            pl.semaphore_signal(
                barrier_sem, inc=INC,
                device_id=(up,), device_id_type=pl.DeviceIdType.MESH,
            )

        @pl.when(has_down)
        def _():
            pl.semaphore_signal(
                barrier_sem, inc=INC,
                device_id=(down,), device_id_type=pl.DeviceIdType.MESH,
            )

        pl.semaphore_wait(barrier_sem, 2 * INC)

        @pl.when(has_up)
        def _():
            pl.semaphore_wait(barrier_sem, INC)

        @pl.when(has_down)
        def _():
            pl.semaphore_wait(barrier_sem, INC)

        def half_at(s, hf):
            return (pl.ds(s * m_per + hf * m_half, m_half), slice(None))

        def rdma(src, dst_slot, hf, ssem, rsem, dev):
            return pltpu.make_async_remote_copy(
                src_ref=src,
                dst_ref=xg_ref.at[half_at(dst_slot, hf)],
                send_sem=ssem,
                recv_sem=rsem,
                device_id=(dev,),
                device_id_type=pl.DeviceIdType.MESH,
            )

        sends = []

        def cstart(cond, rd):
            if cond is None:
                rd.start()
            else:
                @pl.when(cond)
                def _():
                    rd.start()
            sends.append((cond, rd))

        def cwait_recv(cond, rd):
            if cond is None:
                rd.wait_recv()
            else:
                @pl.when(cond)
                def _():
                    rd.wait_recv()

        own_slot = q * N_Z + p
        xg_ref[pl.ds(own_slot * m_per, m_per), :] = x_ref[...]

        def x_half(hf):
            return x_ref.at[pl.ds(hf * m_half, m_half), :]

        def xg_half(s, hf):
            return xg_ref.at[half_at(s, hf)]

        for hf in range(N_H):
            cstart(has_up, rdma(x_half(hf), own_slot, hf,
                                su.at[hf], ru.at[hf], up))
            cstart(has_down, rdma(x_half(hf), own_slot, hf,
                                  sd.at[hf], rdn.at[hf], down))
            cstart(None, rdma(x_half(hf), own_slot, hf,
                              sR.at[p * N_H + hf], rR.at[p * N_H + hf],
                              right_dev))
            cstart(None, rdma(x_half(hf), own_slot, hf,
                              sL.at[p * N_H + hf], rL.at[p * N_H + hf],
                              left_dev))

        for d in (1, 2, 3):
            ju = p - d
            cond_u = p >= d
            sj = q * N_Z + ju
            for hf in range(N_H):
                ix = (d - 1) * N_H + hf
                cwait_recv(cond_u, rdma(x_half(hf), sj, hf,
                                        ru.at[ix], ru.at[ix], up))
                if d <= 2:
                    cstart(jnp.logical_and(cond_u, has_up),
                           rdma(xg_half(sj, hf), sj, hf,
                                su.at[d * N_H + hf], ru.at[d * N_H + hf], up))
                cstart(cond_u, rdma(xg_half(sj, hf), sj, hf,
                                    sR.at[ju * N_H + hf],
                                    rR.at[ju * N_H + hf], right_dev))
                cstart(cond_u, rdma(xg_half(sj, hf), sj, hf,
                                    sL.at[ju * N_H + hf],
                                    rL.at[ju * N_H + hf], left_dev))

            jd = p + d
            cond_d = jd <= N_Z - 1
            sj = q * N_Z + jd
            for hf in range(N_H):
                ix = (d - 1) * N_H + hf
                cwait_recv(cond_d, rdma(x_half(hf), sj, hf,
                                        rdn.at[ix], rdn.at[ix], up))
                if d <= 2:
                    cstart(jnp.logical_and(cond_d, has_down),
                           rdma(xg_half(sj, hf), sj, hf,
                                sd.at[d * N_H + hf], rdn.at[d * N_H + hf],
                                down))
                cstart(cond_d, rdma(xg_half(sj, hf), sj, hf,
                                    sR.at[jd * N_H + hf],
                                    rR.at[jd * N_H + hf], right_dev))
                cstart(cond_d, rdma(xg_half(sj, hf), sj, hf,
                                    sL.at[jd * N_H + hf],
                                    rL.at[jd * N_H + hf], left_dev))

        PER_R = N_Z * N_H

        def for_each_chunk(f):
            f(None, p)
            for d in (1, 2, 3):
                f(p >= d, p - d)
                f(p + d <= N_Z - 1, p + d)

        for h in (1, 2, 3):
            q_or = ring_q(lax.rem(r - h + 8, 8))
            q_ol = ring_q(lax.rem(r + h, 8)) if h <= 2 else None

            def fwd(cond, j, h=h, q_or=q_or, q_ol=q_ol):
                sj = q_or * N_Z + j
                for hf in range(N_H):
                    ix = j * N_H + hf
                    cwait_recv(cond, rdma(x_half(hf), sj, hf,
                                          rR.at[(h - 1) * PER_R + ix],
                                          rR.at[(h - 1) * PER_R + ix],
                                          right_dev))
                    cstart(cond, rdma(xg_half(sj, hf), sj, hf,
                                      sR.at[h * PER_R + ix],
                                      rR.at[h * PER_R + ix], right_dev))
                if q_ol is not None:
                    sl = q_ol * N_Z + j
                    for hf in range(N_H):
                        ix = j * N_H + hf
                        cwait_recv(cond, rdma(x_half(hf), sl, hf,
                                              rL.at[(h - 1) * PER_R + ix],
                                              rL.at[(h - 1) * PER_R + ix],
                                              left_dev))
                        cstart(cond, rdma(xg_half(sl, hf), sl, hf,
                                          sL.at[h * PER_R + ix],
                                          rL.at[h * PER_R + ix], left_dev))

            for_each_chunk(fwd)

        q_o4 = ring_q(lax.rem(r + 4, 8))
        q_o3l = ring_q(lax.rem(r + 3, 8))

        def last(cond, j):
            for hf in range(N_H):
                ix = j * N_H + hf
                cwait_recv(cond, rdma(x_half(hf), q_o4 * N_Z + j, hf,
                                      rR.at[3 * PER_R + ix],
                                      rR.at[3 * PER_R + ix], right_dev))
                cwait_recv(cond, rdma(x_half(hf), q_o3l * N_Z + j, hf,
                                      rL.at[2 * PER_R + ix],
                                      rL.at[2 * PER_R + ix], left_dev))

        for_each_chunk(last)

        for cond, rd in sends:
            if cond is None:
                rd.wait_send()
            else:
                @pl.when(cond)
                def _(rd=rd):
                    rd.wait_send()

        y = jnp.dot(xg_ref[...], w_ref[...], preferred_element_type=jnp.float32)
        c = 0.7978845608028654
        g = 0.5 * y * (1.0 + jnp.tanh(c * (y + 0.044715 * y * y * y)))
        for ell in range(N_DEV):
            s = (ell % N_PLANE) * N_Z + ell // N_PLANE
            out_ref[ell * m_per:(ell + 1) * m_per, :] = \
                g[s * m_per:(s + 1) * m_per, :]

    return pl.pallas_call(
        body,
        out_shape=jax.ShapeDtypeStruct((M, n_per), jnp.float32),
        in_specs=[
            pl.BlockSpec(memory_space=pltpu.VMEM),
            pl.BlockSpec(memory_space=pltpu.VMEM),
        ],
        out_specs=pl.BlockSpec(memory_space=pltpu.VMEM),
        scratch_shapes=[
            pltpu.VMEM((M, k), jnp.float32),
            pltpu.SemaphoreType.DMA((3 * N_H,)),
            pltpu.SemaphoreType.DMA((3 * N_H,)),
            pltpu.SemaphoreType.DMA((3 * N_H,)),
            pltpu.SemaphoreType.DMA((3 * N_H,)),
            pltpu.SemaphoreType.DMA((16 * N_H,)),
            pltpu.SemaphoreType.DMA((16 * N_H,)),
            pltpu.SemaphoreType.DMA((12 * N_H,)),
            pltpu.SemaphoreType.DMA((12 * N_H,)),
        ],
        compiler_params=pltpu.CompilerParams(collective_id=0),
    )(x, w_mat)


# device time: 33499 ns/iter; 2.0187x vs baseline; 1.0760x over previous
import jax
import jax.numpy as jnp
from jax import lax
from jax.experimental import pallas as pl
from jax.experimental.pallas import tpu as pltpu

N_DEV = 32
N_PLANE = 8
N_Z = 4
N_H = 2


def kernel(x, w_mat):
    m_per, k = x.shape
    _, n_per = w_mat.shape
    M = N_DEV * m_per
    m_half = m_per // N_H

    def body(x_ref, w_ref, out_ref, xg_ref,
             su, ru, sd, rdn, sR, rR, sL, rL):
        me = lax.axis_index("i")
        p = me // N_PLANE
        q = lax.rem(me, N_PLANE)
        base = me - q

        def ring_q(rr):
            return jnp.where(rr < 3, rr, jnp.where(rr < 6, rr + 2, 10 - rr))

        r = jnp.where(q < 3, q, jnp.where(q >= 5, q - 2,
                                          jnp.where(q == 4, 6, 7)))
        right_dev = base + ring_q(lax.rem(r + 1, 8))
        left_dev = base + ring_q(lax.rem(r + 7, 8))
        up = me + N_PLANE
        down = me - N_PLANE
        has_up = p < N_Z - 1
        has_down = p > 0

        INC = 1 << 15
        barrier_sem = pltpu.get_barrier_semaphore()
        for nbr in (left_dev, right_dev):
            pl.semaphore_signal(
                barrier_sem, inc=INC,
                device_id=(nbr,), device_id_type=pl.DeviceIdType.MESH,
            )

        @pl.when(has_up)
        def _():
            pl.semaphore_signal(
                barrier_sem, inc=INC,
                device_id=(up,), device_id_type=pl.DeviceIdType.MESH,
            )

        @pl.when(has_down)
        def _():
            pl.semaphore_signal(
                barrier_sem, inc=INC,
                device_id=(down,), device_id_type=pl.DeviceIdType.MESH,
            )

        pl.semaphore_wait(barrier_sem, 2 * INC)

        @pl.when(has_up)
        def _():
            pl.semaphore_wait(barrier_sem, INC)

        @pl.when(has_down)
        def _():
            pl.semaphore_wait(barrier_sem, INC)

        def half_at(s, hf):
            return (pl.ds(s * m_per + hf * m_half, m_half), slice(None))

        def rdma(src, dst_slot, hf, ssem, rsem, dev):
            return pltpu.make_async_remote_copy(
                src_ref=src,
                dst_ref=xg_ref.at[half_at(dst_slot, hf)],
                send_sem=ssem,
                recv_sem=rsem,
                device_id=(dev,),
                device_id_type=pl.DeviceIdType.MESH,
            )

        sends = []

        def cstart(cond, rd):
            if cond is None:
                rd.start()
            else:
                @pl.when(cond)
                def _():
                    rd.start()
            sends.append((cond, rd))

        def cwait_recv(cond, rd):
            if cond is None:
                rd.wait_recv()
            else:
                @pl.when(cond)
                def _():
                    rd.wait_recv()

        own_slot = q * N_Z + p
        xg_ref[pl.ds(own_slot * m_per, m_per), :] = x_ref[...]

        def x_half(hf):
            return x_ref.at[pl.ds(hf * m_half, m_half), :]

        def xg_half(s, hf):
            return xg_ref.at[half_at(s, hf)]

        for hf in range(N_H):
            cstart(has_up, rdma(x_half(hf), own_slot, hf,
                                su.at[hf], ru.at[hf], up))
            cstart(has_down, rdma(x_half(hf), own_slot, hf,
                                  sd.at[hf], rdn.at[hf], down))
            cstart(None, rdma(x_half(hf), own_slot, hf,
                              sR.at[p * N_H + hf], rR.at[p * N_H + hf],
                              right_dev))
            cstart(None, rdma(x_half(hf), own_slot, hf,
                              sL.at[p * N_H + hf], rL.at[p * N_H + hf],
                              left_dev))

        for d in (1, 2, 3):
            ju = p - d
            cond_u = p >= d
            sj = q * N_Z + ju
            for hf in range(N_H):
                ix = (d - 1) * N_H + hf
                cwait_recv(cond_u, rdma(x_half(hf), sj, hf,
                                        ru.at[ix], ru.at[ix], up))
                if d <= 2:
                    cstart(jnp.logical_and(cond_u, has_up),
                           rdma(xg_half(sj, hf), sj, hf,
                                su.at[d * N_H + hf], ru.at[d * N_H + hf], up))
                cstart(cond_u, rdma(xg_half(sj, hf), sj, hf,
                                    sR.at[ju * N_H + hf],
                                    rR.at[ju * N_H + hf], right_dev))
                cstart(cond_u, rdma(xg_half(sj, hf), sj, hf,
                                    sL.at[ju * N_H + hf],
                                    rL.at[ju * N_H + hf], left_dev))

            jd = p + d
            cond_d = jd <= N_Z - 1
            sj = q * N_Z + jd
            for hf in range(N_H):
                ix = (d - 1) * N_H + hf
                cwait_recv(cond_d, rdma(x_half(hf), sj, hf,
                                        rdn.at[ix], rdn.at[ix], up))
                if d <= 2:
                    cstart(jnp.logical_and(cond_d, has_down),
                           rdma(xg_half(sj, hf), sj, hf,
                                sd.at[d * N_H + hf], rdn.at[d * N_H + hf],
                                down))
                cstart(cond_d, rdma(xg_half(sj, hf), sj, hf,
                                    sR.at[jd * N_H + hf],
                                    rR.at[jd * N_H + hf], right_dev))
                cstart(cond_d, rdma(xg_half(sj, hf), sj, hf,
                                    sL.at[jd * N_H + hf],
                                    rL.at[jd * N_H + hf], left_dev))

        PER_R = N_Z * N_H

        def for_each_chunk(f):
            f(None, p)
            for d in (1, 2, 3):
                f(p >= d, p - d)
                f(p + d <= N_Z - 1, p + d)

        def band(cond, b):
            return b if cond is None else jnp.logical_and(cond, b)

        for h in (1, 2, 3):
            q_or = ring_q(lax.rem(r - h + 8, 8))
            q_ol = ring_q(lax.rem(r + h, 8))

            def fwd(cond, j, h=h, q_or=q_or, q_ol=q_ol):
                sj = q_or * N_Z + j
                for hf in range(N_H):
                    ix = j * N_H + hf
                    cwait_recv(cond, rdma(x_half(hf), sj, hf,
                                          rR.at[(h - 1) * PER_R + ix],
                                          rR.at[(h - 1) * PER_R + ix],
                                          right_dev))
                    cstart(cond if h < 3 else band(cond, j < 2),
                           rdma(xg_half(sj, hf), sj, hf,
                                sR.at[h * PER_R + ix],
                                rR.at[h * PER_R + ix], right_dev))
                sl = q_ol * N_Z + j
                for hf in range(N_H):
                    ix = j * N_H + hf
                    if h < 3:
                        cwait_recv(cond, rdma(x_half(hf), sl, hf,
                                              rL.at[(h - 1) * PER_R + ix],
                                              rL.at[(h - 1) * PER_R + ix],
                                              left_dev))
                        cstart(cond, rdma(xg_half(sl, hf), sl, hf,
                                          sL.at[h * PER_R + ix],
                                          rL.at[h * PER_R + ix], left_dev))
                    else:
                        cwait_recv(cond, rdma(x_half(hf), sl, hf,
                                              rL.at[2 * PER_R + ix],
                                              rL.at[2 * PER_R + ix],
                                              left_dev))
                        cstart(band(cond, j >= 2),
                               rdma(xg_half(sl, hf), sl, hf,
                                    sL.at[3 * PER_R + ix],
                                    rL.at[3 * PER_R + ix], left_dev))

            for_each_chunk(fwd)

        q_o4 = ring_q(lax.rem(r + 4, 8))

        def last(cond, j):
            for hf in range(N_H):
                ix = j * N_H + hf
                cwait_recv(band(cond, j < 2),
                           rdma(x_half(hf), q_o4 * N_Z + j, hf,
                                rR.at[3 * PER_R + ix],
                                rR.at[3 * PER_R + ix], right_dev))
                cwait_recv(band(cond, j >= 2),
                           rdma(x_half(hf), q_o4 * N_Z + j, hf,
                                rL.at[3 * PER_R + ix],
                                rL.at[3 * PER_R + ix], left_dev))

        for_each_chunk(last)

        for cond, rd in sends:
            if cond is None:
                rd.wait_send()
            else:
                @pl.when(cond)
                def _(rd=rd):
                    rd.wait_send()

        y = jnp.dot(xg_ref[...], w_ref[...], preferred_element_type=jnp.float32)
        c = 0.7978845608028654
        g = 0.5 * y * (1.0 + jnp.tanh(c * (y + 0.044715 * y * y * y)))
        for ell in range(N_DEV):
            s = (ell % N_PLANE) * N_Z + ell // N_PLANE
            out_ref[ell * m_per:(ell + 1) * m_per, :] = \
                g[s * m_per:(s + 1) * m_per, :]

    return pl.pallas_call(
        body,
        out_shape=jax.ShapeDtypeStruct((M, n_per), jnp.float32),
        in_specs=[
            pl.BlockSpec(memory_space=pltpu.VMEM),
            pl.BlockSpec(memory_space=pltpu.VMEM),
        ],
        out_specs=pl.BlockSpec(memory_space=pltpu.VMEM),
        scratch_shapes=[
            pltpu.VMEM((M, k), jnp.float32),
            pltpu.SemaphoreType.DMA((3 * N_H,)),
            pltpu.SemaphoreType.DMA((3 * N_H,)),
            pltpu.SemaphoreType.DMA((3 * N_H,)),
            pltpu.SemaphoreType.DMA((3 * N_H,)),
            pltpu.SemaphoreType.DMA((16 * N_H,)),
            pltpu.SemaphoreType.DMA((16 * N_H,)),
            pltpu.SemaphoreType.DMA((16 * N_H,)),
            pltpu.SemaphoreType.DMA((16 * N_H,)),
        ],
        compiler_params=pltpu.CompilerParams(collective_id=0),
    )(x, w_mat)


# device time: 33018 ns/iter; 2.0481x vs baseline; 1.0146x over previous
import jax
import jax.numpy as jnp
from jax import lax
from jax.experimental import pallas as pl
from jax.experimental.pallas import tpu as pltpu

N_DEV = 32
N_PLANE = 8
N_Z = 4
N_H = 2


def kernel(x, w_mat):
    m_per, k = x.shape
    _, n_per = w_mat.shape
    M = N_DEV * m_per
    m_half = m_per // N_H

    def body(x_ref, w_ref, out_ref, xg_ref,
             su, ru, sd, rdn, sR, rR, sL, rL):
        me = lax.axis_index("i")
        p = me // N_PLANE
        q = lax.rem(me, N_PLANE)
        base = me - q

        def ring_q(rr):
            return jnp.where(rr < 3, rr, jnp.where(rr < 6, rr + 2, 10 - rr))

        r = jnp.where(q < 3, q, jnp.where(q >= 5, q - 2,
                                          jnp.where(q == 4, 6, 7)))
        right_dev = base + ring_q(lax.rem(r + 1, 8))
        left_dev = base + ring_q(lax.rem(r + 7, 8))
        up = me + N_PLANE
        down = me - N_PLANE
        has_up = p < N_Z - 1
        has_down = p > 0

        INC = 1 << 15
        barrier_sem = pltpu.get_barrier_semaphore()
        for nbr in (left_dev, right_dev):
            pl.semaphore_signal(
                barrier_sem, inc=INC,
                device_id=(nbr,), device_id_type=pl.DeviceIdType.MESH,
            )

        @pl.when(has_up)
        def _():
            pl.semaphore_signal(
                barrier_sem, inc=INC,
                device_id=(up,), device_id_type=pl.DeviceIdType.MESH,
            )

        @pl.when(has_down)
        def _():
            pl.semaphore_signal(
                barrier_sem, inc=INC,
                device_id=(down,), device_id_type=pl.DeviceIdType.MESH,
            )

        pl.semaphore_wait(barrier_sem, 2 * INC)

        @pl.when(has_up)
        def _():
            pl.semaphore_wait(barrier_sem, INC)

        @pl.when(has_down)
        def _():
            pl.semaphore_wait(barrier_sem, INC)

        def half_at(s, hf):
            return (pl.ds(s * m_per + hf * m_half, m_half), slice(None))

        def rdma(src, dst_slot, hf, ssem, rsem, dev):
            return pltpu.make_async_remote_copy(
                src_ref=src,
                dst_ref=xg_ref.at[half_at(dst_slot, hf)],
                send_sem=ssem,
                recv_sem=rsem,
                device_id=(dev,),
                device_id_type=pl.DeviceIdType.MESH,
            )

        sends = []

        def cstart(cond, rd):
            if cond is None:
                rd.start()
            else:
                @pl.when(cond)
                def _():
                    rd.start()
            sends.append((cond, rd))

        def cwait_recv(cond, rd):
            if cond is None:
                rd.wait_recv()
            else:
                @pl.when(cond)
                def _():
                    rd.wait_recv()

        own_slot = q * N_Z + p
        xg_ref[pl.ds(own_slot * m_per, m_per), :] = x_ref[...]

        def x_half(hf):
            return x_ref.at[pl.ds(hf * m_half, m_half), :]

        def xg_half(s, hf):
            return xg_ref.at[half_at(s, hf)]

        for hf in range(N_H):
            cstart(has_up, rdma(x_half(hf), own_slot, hf,
                                su.at[hf], ru.at[hf], up))
            cstart(has_down, rdma(x_half(hf), own_slot, hf,
                                  sd.at[hf], rdn.at[hf], down))
            cstart(None, rdma(x_half(hf), own_slot, hf,
                              sR.at[p * N_H + hf], rR.at[p * N_H + hf],
                              right_dev))
            cstart(None, rdma(x_half(hf), own_slot, hf,
                              sL.at[p * N_H + hf], rL.at[p * N_H + hf],
                              left_dev))

        for d in (1, 2, 3):
            ju = p - d
            cond_u = p >= d
            sj = q * N_Z + ju
            for hf in range(N_H):
                ix = (d - 1) * N_H + hf
                cwait_recv(cond_u, rdma(x_half(hf), sj, hf,
                                        ru.at[ix], ru.at[ix], up))
                if d <= 2:
                    cstart(jnp.logical_and(cond_u, has_up),
                           rdma(xg_half(sj, hf), sj, hf,
                                su.at[d * N_H + hf], ru.at[d * N_H + hf], up))
                cstart(cond_u, rdma(xg_half(sj, hf), sj, hf,
                                    sR.at[ju * N_H + hf],
                                    rR.at[ju * N_H + hf], right_dev))
                cstart(cond_u, rdma(xg_half(sj, hf), sj, hf,
                                    sL.at[ju * N_H + hf],
                                    rL.at[ju * N_H + hf], left_dev))

            jd = p + d
            cond_d = jd <= N_Z - 1
            sj = q * N_Z + jd
            for hf in range(N_H):
                ix = (d - 1) * N_H + hf
                cwait_recv(cond_d, rdma(x_half(hf), sj, hf,
                                        rdn.at[ix], rdn.at[ix], up))
                if d <= 2:
                    cstart(jnp.logical_and(cond_d, has_down),
                           rdma(xg_half(sj, hf), sj, hf,
                                sd.at[d * N_H + hf], rdn.at[d * N_H + hf],
                                down))
                cstart(cond_d, rdma(xg_half(sj, hf), sj, hf,
                                    sR.at[jd * N_H + hf],
                                    rR.at[jd * N_H + hf], right_dev))
                cstart(cond_d, rdma(xg_half(sj, hf), sj, hf,
                                    sL.at[jd * N_H + hf],
                                    rL.at[jd * N_H + hf], left_dev))

        PER_R = N_Z * N_H

        def for_each_chunk(f):
            f(None, p)
            for d in (1, 2, 3):
                f(p >= d, p - d)
                f(p + d <= N_Z - 1, p + d)

        def band(cond, b):
            return b if cond is None else jnp.logical_and(cond, b)

        for h in (1, 2, 3):
            q_or = ring_q(lax.rem(r - h + 8, 8))
            q_ol = ring_q(lax.rem(r + h, 8))

            def fwd(cond, j, h=h, q_or=q_or, q_ol=q_ol):
                sj = q_or * N_Z + j
                for hf in range(N_H):
                    ix = j * N_H + hf
                    cwait_recv(cond, rdma(x_half(hf), sj, hf,
                                          rR.at[(h - 1) * PER_R + ix],
                                          rR.at[(h - 1) * PER_R + ix],
                                          right_dev))
                    cstart(cond if h < 3 else band(cond, j < 2),
                           rdma(xg_half(sj, hf), sj, hf,
                                sR.at[h * PER_R + ix],
                                rR.at[h * PER_R + ix], right_dev))
                sl = q_ol * N_Z + j
                for hf in range(N_H):
                    ix = j * N_H + hf
                    if h < 3:
                        cwait_recv(cond, rdma(x_half(hf), sl, hf,
                                              rL.at[(h - 1) * PER_R + ix],
                                              rL.at[(h - 1) * PER_R + ix],
                                              left_dev))
                        cstart(cond, rdma(xg_half(sl, hf), sl, hf,
                                          sL.at[h * PER_R + ix],
                                          rL.at[h * PER_R + ix], left_dev))
                    else:
                        cwait_recv(cond, rdma(x_half(hf), sl, hf,
                                              rL.at[2 * PER_R + ix],
                                              rL.at[2 * PER_R + ix],
                                              left_dev))
                        cstart(band(cond, j >= 2),
                               rdma(xg_half(sl, hf), sl, hf,
                                    sL.at[3 * PER_R + ix],
                                    rL.at[3 * PER_R + ix], left_dev))

            for_each_chunk(fwd)

        q_o4 = ring_q(lax.rem(r + 4, 8))

        def gelu(y):
            c = 0.7978845608028654
            return 0.5 * y * (1.0 + jnp.tanh(c * (y + 0.044715 * y * y * y)))

        y = jnp.dot(xg_ref[...], w_ref[...], preferred_element_type=jnp.float32)
        g = gelu(y)
        for ell in range(N_DEV):
            s = (ell % N_PLANE) * N_Z + ell // N_PLANE
            out_ref[ell * m_per:(ell + 1) * m_per, :] = \
                g[s * m_per:(s + 1) * m_per, :]


        def last(cond, j):
            for hf in range(N_H):
                ix = j * N_H + hf
                cwait_recv(band(cond, j < 2),
                           rdma(x_half(hf), q_o4 * N_Z + j, hf,
                                rR.at[3 * PER_R + ix],
                                rR.at[3 * PER_R + ix], right_dev))
                cwait_recv(band(cond, j >= 2),
                           rdma(x_half(hf), q_o4 * N_Z + j, hf,
                                rL.at[3 * PER_R + ix],
                                rL.at[3 * PER_R + ix], left_dev))

        for_each_chunk(last)

        for cond, rd in sends:
            if cond is None:
                rd.wait_send()
            else:
                @pl.when(cond)
                def _(rd=rd):
                    rd.wait_send()

        yb = jnp.dot(xg_ref[pl.ds(q_o4 * N_Z * m_per, N_Z * m_per), :],
                     w_ref[...], preferred_element_type=jnp.float32)
        gb = gelu(yb)
        for j in range(N_Z):
            out_ref[pl.ds((j * N_PLANE + q_o4) * m_per, m_per), :] = \
                gb[j * m_per:(j + 1) * m_per, :]

    return pl.pallas_call(
        body,
        out_shape=jax.ShapeDtypeStruct((M, n_per), jnp.float32),
        in_specs=[
            pl.BlockSpec(memory_space=pltpu.VMEM),
            pl.BlockSpec(memory_space=pltpu.VMEM),
        ],
        out_specs=pl.BlockSpec(memory_space=pltpu.VMEM),
        scratch_shapes=[
            pltpu.VMEM((M, k), jnp.float32),
            pltpu.SemaphoreType.DMA((3 * N_H,)),
            pltpu.SemaphoreType.DMA((3 * N_H,)),
            pltpu.SemaphoreType.DMA((3 * N_H,)),
            pltpu.SemaphoreType.DMA((3 * N_H,)),
            pltpu.SemaphoreType.DMA((16 * N_H,)),
            pltpu.SemaphoreType.DMA((16 * N_H,)),
            pltpu.SemaphoreType.DMA((16 * N_H,)),
            pltpu.SemaphoreType.DMA((16 * N_H,)),
        ],
        compiler_params=pltpu.CompilerParams(collective_id=0),
    )(x, w_mat)
